# baseline (speedup 1.0000x reference)
import functools

import jax
import jax.numpy as jnp
from jax import lax
from jax.experimental import pallas as pl
from jax.experimental.pallas import tpu as pltpu
from jax.experimental.pallas import tpu_sc as plsc

N_NODES = 10000
N_EDGES = 320000
D_NODE = 128
D_EDGE = 16
D_U = 128
HID = 512
D_OUT = 128

_C = 5
_EC = N_EDGES // _C

BE = 2560
BN = 2000

_NC = 2
_NS = 16
_NW = _NC * _NS
_EPW = _EC // _NW
_GCH = 80
_NCH = _EPW // _GCH
_GK = 5
_GROWS = _GK * _GCH
_GOUT = _EPW // _GROWS

_SC_MESH = plsc.VectorSubcoreMesh(core_axis_name="c", subcore_axis_name="s")


_NPAD = 10240
_NPT = _NPAD // _NS


@functools.partial(
    pl.kernel,
    out_type=[jax.ShapeDtypeStruct((_EC, D_NODE), jnp.float32),
              jax.ShapeDtypeStruct((_EC, D_NODE), jnp.float32)],
    mesh=_SC_MESH,
    scratch_types=[
        pltpu.VMEM((_EPW,), jnp.int32),
        pltpu.VMEM((_EPW,), jnp.int32),
        pltpu.VMEM((_GROWS, D_NODE), jnp.float32),
        pltpu.VMEM((_GROWS, D_NODE), jnp.float32),
        pltpu.SemaphoreType.DMA,
        pltpu.SemaphoreType.DMA,
    ],
)
def _sc_gather(node_hbm, src_hbm, dst_hbm, gs_hbm, gd_hbm,
               src_v, dst_v, rs_v, rd_v, sem_g, sem_w):
    wid = lax.axis_index("s") * _NC + lax.axis_index("c")
    base = wid * _EPW
    pltpu.sync_copy(src_hbm.at[pl.ds(base, _EPW)], src_v)
    pltpu.sync_copy(dst_hbm.at[pl.ds(base, _EPW)], dst_v)

    def body(j, carry):
        off = j * _GROWS
        copies = []
        for b in range(_GK):
            o = b * _GCH
            copies.append(pltpu.async_copy(
                node_hbm.at[src_v.at[pl.ds(off + o, _GCH)]],
                rs_v.at[pl.ds(o, _GCH)], sem_g))
            copies.append(pltpu.async_copy(
                node_hbm.at[dst_v.at[pl.ds(off + o, _GCH)]],
                rd_v.at[pl.ds(o, _GCH)], sem_g))
        for c in copies:
            c.wait()
        ws = pltpu.async_copy(rs_v, gs_hbm.at[pl.ds(base + off, _GROWS)],
                              sem_w)
        wd = pltpu.async_copy(rd_v, gd_hbm.at[pl.ds(base + off, _GROWS)],
                              sem_w)
        ws.wait()
        wd.wait()
        return carry

    lax.fori_loop(0, _GOUT, body, 0)


def _scatter_work(e_refs, dst4_ref, zeros_ref, hp_ref, idx_v, rows_v, acc_sh):
    k = len(e_refs)
    cid = lax.axis_index("c")
    sid = lax.axis_index("s")
    wid = sid * _NC + cid
    base = wid * _EPW
    row0 = sid * _NPT
    pltpu.sync_copy(zeros_ref.at[pl.ds(row0, _NPT)],
                    acc_sh.at[pl.ds(row0, _NPT)])
    for ci in range(k):
        pltpu.sync_copy(dst4_ref.at[ci, wid],
                        idx_v.at[pl.ds(ci * _NCH, _NCH)])
    plsc.subcore_barrier()

    for ci in range(k):
        def body(j, carry):
            pltpu.sync_copy(e_refs[ci].at[pl.ds(base + j * _GCH, _GCH)],
                            rows_v)
            pltpu.sync_copy(rows_v, acc_sh.at[idx_v.at[ci * _NCH + j]],
                            add=True)
            return carry

        lax.fori_loop(0, _NCH, body, 0)

    plsc.subcore_barrier()
    pltpu.sync_copy(acc_sh.at[pl.ds(row0, _NPT)],
                    hp_ref.at[cid, pl.ds(row0, _NPT)])


def _scatter_types(k):
    return dict(
        out_type=jax.ShapeDtypeStruct((_NC, _NPAD, D_NODE), jnp.float32),
        mesh=_SC_MESH,
        scratch_types=[
            pltpu.VMEM((k * _NCH, _GCH), jnp.int32),
            pltpu.VMEM((_GCH, D_NODE), jnp.float32),
            pltpu.VMEM_SHARED((_NPAD, D_NODE), jnp.float32),
        ],
    )


@functools.partial(pl.kernel, **_scatter_types(1))
def _sc_scatter1(e0, dst4, zeros_hbm, prev_hbm, hp, idx_v, rows_v, acc_sh):
    _scatter_work([e0], dst4, zeros_hbm, hp, idx_v, rows_v, acc_sh)


def _edge_body(ef_ref, gs_ref, gd_ref, wa_ref, wb_ref, wc_ref, ce_ref,
               w2_ref, b2_ref, e_ref, ecomb_ref):
    i = pl.program_id(0)
    bf = jnp.bfloat16
    h = jnp.dot(ef_ref[...].astype(bf), wa_ref[...],
                preferred_element_type=jnp.float32)
    h += jnp.dot(gs_ref[...].astype(bf), wb_ref[...],
                 preferred_element_type=jnp.float32)
    h += jnp.dot(gd_ref[...].astype(bf), wc_ref[...],
                 preferred_element_type=jnp.float32)
    h += ce_ref[...]
    h = jnp.maximum(h, 0.0).astype(bf)
    e = jnp.dot(h, w2_ref[...], preferred_element_type=jnp.float32)
    e += b2_ref[...]
    e_ref[...] = e

    @pl.when(i == 0)
    def _():
        ecomb_ref[...] = jnp.zeros_like(ecomb_ref)

    ecomb_ref[...] += jnp.sum(e, axis=0, keepdims=True)


def _node_body(nf_ref, hp0_ref, hp1_ref, hp2_ref, hp3_ref, hp4_ref,
               ecomb_ref, wnn_ref, wnh_ref, cn_ref,
               wn2_ref, bn2_ref, wgn_ref, wge_ref, cu_ref, wu2_ref, bu2_ref,
               n_ref, g_ref, nacc_ref):
    i = pl.program_id(0)
    bf = jnp.bfloat16
    hsum = (hp0_ref[0] + hp0_ref[1] + hp1_ref[0] + hp1_ref[1]
            + hp2_ref[0] + hp2_ref[1] + hp3_ref[0] + hp3_ref[1]
            + hp4_ref[0] + hp4_ref[1])
    x = jnp.dot(nf_ref[...].astype(bf), wnn_ref[...],
                preferred_element_type=jnp.float32)
    x += jnp.dot(hsum.astype(bf), wnh_ref[...],
                 preferred_element_type=jnp.float32)
    x += cn_ref[...]
    x = jnp.maximum(x, 0.0).astype(bf)
    n = jnp.dot(x, wn2_ref[...], preferred_element_type=jnp.float32)
    n += bn2_ref[...]
    n_ref[...] = n

    @pl.when(i == 0)
    def _():
        nacc_ref[...] = jnp.zeros_like(nacc_ref)

    nacc_ref[...] += jnp.sum(n, axis=0, keepdims=True)

    @pl.when(i == pl.num_programs(0) - 1)
    def _():
        g1 = jnp.dot(nacc_ref[...], wgn_ref[...],
                     preferred_element_type=jnp.float32)
        g1 += jnp.dot(ecomb_ref[...], wge_ref[...],
                      preferred_element_type=jnp.float32)
        g1 += cu_ref[...]
        g1 = jnp.maximum(g1, 0.0)
        g_ref[...] = jnp.dot(g1, wu2_ref[...],
                             preferred_element_type=jnp.float32) + bu2_ref[...]


def _const_spec(shape):
    return pl.BlockSpec(shape, lambda i: tuple(0 for _ in shape))


def _edge_mlp(edge_feat, gs, gd, wa, wb, wc, ce, w2, b2):
    ne = edge_feat.shape[0]
    grid = ne // BE
    return pl.pallas_call(
        _edge_body,
        grid=(grid,),
        in_specs=[
            pl.BlockSpec((BE, D_EDGE), lambda i: (i, 0)),
            pl.BlockSpec((BE, D_NODE), lambda i: (i, 0)),
            pl.BlockSpec((BE, D_NODE), lambda i: (i, 0)),
            _const_spec((D_EDGE, HID)),
            _const_spec((D_NODE, HID)),
            _const_spec((D_NODE, HID)),
            _const_spec((1, HID)),
            _const_spec((HID, D_OUT)),
            _const_spec((1, D_OUT)),
        ],
        out_specs=[
            pl.BlockSpec((BE, D_OUT), lambda i: (i, 0)),
            pl.BlockSpec((1, D_OUT), lambda i: (0, 0)),
        ],
        out_shape=[
            jax.ShapeDtypeStruct((ne, D_OUT), jnp.float32),
            jax.ShapeDtypeStruct((1, D_OUT), jnp.float32),
        ],
    )(edge_feat, gs, gd, wa, wb, wc, ce, w2, b2)


def _node_mlp(node_feat, hps, ecomb, wnn, wnh, cn, wn2, bn2,
              wgn, wge, cu, wu2, bu2):
    nn = node_feat.shape[0]
    grid = nn // BN
    hp_spec = pl.BlockSpec((_NC, BN, D_NODE), lambda i: (0, i, 0))
    return pl.pallas_call(
        _node_body,
        grid=(grid,),
        in_specs=[
            pl.BlockSpec((BN, D_NODE), lambda i: (i, 0)),
            hp_spec, hp_spec, hp_spec, hp_spec, hp_spec,
            _const_spec((1, D_OUT)),
            _const_spec((D_NODE, HID)),
            _const_spec((D_NODE, HID)),
            _const_spec((1, HID)),
            _const_spec((HID, D_OUT)),
            _const_spec((1, D_OUT)),
            _const_spec((D_OUT, HID)),
            _const_spec((D_OUT, HID)),
            _const_spec((1, HID)),
            _const_spec((HID, D_OUT)),
            _const_spec((1, D_OUT)),
        ],
        out_specs=[
            pl.BlockSpec((BN, D_OUT), lambda i: (i, 0)),
            pl.BlockSpec((1, D_OUT), lambda i: (0, 0)),
        ],
        out_shape=[
            jax.ShapeDtypeStruct((nn, D_OUT), jnp.float32),
            jax.ShapeDtypeStruct((1, D_OUT), jnp.float32),
        ],
        scratch_shapes=[pltpu.VMEM((1, D_OUT), jnp.float32)],
    )(node_feat, *hps, ecomb, wnn, wnh, cn, wn2, bn2,
      wgn, wge, cu, wu2, bu2)


def kernel(edge_index, edge_feat, node_feat, g_repr,
           W_edge, b_edge, W_edge2, b_edge2,
           W_node, b_node, W_node2, b_node2,
           W_u, b_u, W_u2, b_u2):
    bf = jnp.bfloat16
    src = edge_index[0]
    dst = edge_index[1]
    u = g_repr

    wa = W_edge[:, :D_EDGE].T.astype(bf)
    wb = W_edge[:, D_EDGE:D_EDGE + D_NODE].T.astype(bf)
    wc = W_edge[:, D_EDGE + D_NODE:D_EDGE + 2 * D_NODE].T.astype(bf)
    ce = (b_edge + u[0] @ W_edge[:, D_EDGE + 2 * D_NODE:].T)[None]
    w2 = W_edge2.T.astype(bf)
    b2 = b_edge2[None]

    wnn = W_node[:, :D_NODE].T.astype(bf)
    wnh = W_node[:, D_NODE:2 * D_NODE].T.astype(bf)
    cn = (b_node + u[0] @ W_node[:, 2 * D_NODE:].T)[None]
    wn2 = W_node2.T.astype(bf)
    bn2 = b_node2[None]

    wgn = W_u[:, :D_OUT].T
    wge = W_u[:, D_OUT:2 * D_OUT].T
    cu = (b_u + u[0] @ W_u[:, 2 * D_OUT:].T)[None]
    wu2 = W_u2.T
    bu2 = b_u2[None]

    zeros_n = jnp.zeros((_NPAD, D_NODE), jnp.float32)

    e_chunks, ecombs, hps = [], [], []
    hp_prev = jnp.zeros((_NC, _NPAD, D_NODE), jnp.float32)
    for c in range(_C):
        sl = slice(c * _EC, (c + 1) * _EC)
        gs, gd = _sc_gather(node_feat, src[sl], dst[sl])
        e_c, ecomb_c = _edge_mlp(edge_feat[sl], gs, gd, wa, wb, wc, ce,
                                 w2, b2)
        dst_t = dst[sl].reshape(1, _NW, _NCH, _GCH)
        hp_prev = _sc_scatter1(e_c, dst_t, zeros_n, hp_prev)
        hps.append(hp_prev)
        e_chunks.append(e_c)
        ecombs.append(ecomb_c)

    e_new = jnp.concatenate(e_chunks, axis=0)
    e_comb = functools.reduce(lambda a, b: a + b, ecombs)

    n_new, g_new = _node_mlp(node_feat, hps, e_comb, wnn, wnh, cn,
                             wn2, bn2, wgn, wge, cu, wu2, bu2)
    return (e_new, n_new, g_new)


# speedup vs baseline: 1.0376x; 1.0376x over previous
import functools

import jax
import jax.numpy as jnp
from jax import lax
from jax.experimental import pallas as pl
from jax.experimental.pallas import tpu as pltpu
from jax.experimental.pallas import tpu_sc as plsc

N_NODES = 10000
N_EDGES = 320000
D_NODE = 128
D_EDGE = 16
D_U = 128
HID = 512
D_OUT = 128

_C = 5
_EC = N_EDGES // _C

BE = 2560
BN = 2000

_NC = 2
_NS = 16
_NW = _NC * _NS
_EPW = _EC // _NW
_GCH = 80
_NCH = _EPW // _GCH
_GK = 5
_GROWS = _GK * _GCH
_GOUT = _EPW // _GROWS

_SC_MESH = plsc.VectorSubcoreMesh(core_axis_name="c", subcore_axis_name="s")


_NPAD = 10240
_NPT = _NPAD // _NS


@functools.partial(
    pl.kernel,
    out_type=jax.ShapeDtypeStruct((_EC, 2 * D_NODE), jnp.float32),
    mesh=_SC_MESH,
    scratch_types=[
        pltpu.VMEM((_EPW,), jnp.int32),
        pltpu.VMEM((_EPW,), jnp.int32),
        pltpu.VMEM((_GROWS, D_NODE), jnp.float32),
        pltpu.VMEM((_GROWS, D_NODE), jnp.float32),
        pltpu.SemaphoreType.DMA,
        pltpu.SemaphoreType.DMA,
    ],
)
def _sc_gather(node_hbm, src_hbm, dst_hbm, g_hbm,
               src_v, dst_v, rs_v, rd_v, sem_g, sem_w):
    wid = lax.axis_index("s") * _NC + lax.axis_index("c")
    base = wid * _EPW
    pltpu.sync_copy(src_hbm.at[pl.ds(base, _EPW)], src_v)
    pltpu.sync_copy(dst_hbm.at[pl.ds(base, _EPW)], dst_v)

    def body(j, carry):
        off = j * _GROWS
        copies = []
        for b in range(_GK):
            o = b * _GCH
            copies.append(pltpu.async_copy(
                node_hbm.at[src_v.at[pl.ds(off + o, _GCH)]],
                rs_v.at[pl.ds(o, _GCH)], sem_g))
            copies.append(pltpu.async_copy(
                node_hbm.at[dst_v.at[pl.ds(off + o, _GCH)]],
                rd_v.at[pl.ds(o, _GCH)], sem_g))
        for c in copies:
            c.wait()
        ws = pltpu.async_copy(
            rs_v, g_hbm.at[pl.ds(base + off, _GROWS), pl.ds(0, D_NODE)],
            sem_w)
        wd = pltpu.async_copy(
            rd_v, g_hbm.at[pl.ds(base + off, _GROWS), pl.ds(D_NODE, D_NODE)],
            sem_w)
        ws.wait()
        wd.wait()
        return carry

    lax.fori_loop(0, _GOUT, body, 0)


def _scatter_work(e_refs, dst4_ref, zeros_ref, hp_ref, idx_v, rows_v, acc_sh):
    k = len(e_refs)
    cid = lax.axis_index("c")
    sid = lax.axis_index("s")
    wid = sid * _NC + cid
    base = wid * _EPW
    row0 = sid * _NPT
    pltpu.sync_copy(zeros_ref.at[pl.ds(row0, _NPT)],
                    acc_sh.at[pl.ds(row0, _NPT)])
    for ci in range(k):
        pltpu.sync_copy(dst4_ref.at[ci, wid],
                        idx_v.at[pl.ds(ci * _NCH, _NCH)])
    plsc.subcore_barrier()

    for ci in range(k):
        def body(j, carry):
            pltpu.sync_copy(e_refs[ci].at[pl.ds(base + j * _GCH, _GCH)],
                            rows_v)
            pltpu.sync_copy(rows_v, acc_sh.at[idx_v.at[ci * _NCH + j]],
                            add=True)
            return carry

        lax.fori_loop(0, _NCH, body, 0)

    plsc.subcore_barrier()
    pltpu.sync_copy(acc_sh.at[pl.ds(row0, _NPT)],
                    hp_ref.at[cid, pl.ds(row0, _NPT)])


def _scatter_types(k):
    return dict(
        out_type=jax.ShapeDtypeStruct((_NC, _NPAD, D_NODE), jnp.float32),
        mesh=_SC_MESH,
        scratch_types=[
            pltpu.VMEM((k * _NCH, _GCH), jnp.int32),
            pltpu.VMEM((_GCH, D_NODE), jnp.float32),
            pltpu.VMEM_SHARED((_NPAD, D_NODE), jnp.float32),
        ],
    )


@functools.partial(pl.kernel, **_scatter_types(1))
def _sc_scatter1(e0, dst4, zeros_hbm, prev_hbm, hp, idx_v, rows_v, acc_sh):
    _scatter_work([e0], dst4, zeros_hbm, hp, idx_v, rows_v, acc_sh)


def _edge_body(ef_ref, g_ref, wa_ref, wbc_ref, ce_ref,
               w2_ref, b2_ref, e_ref, ecomb_ref):
    i = pl.program_id(0)
    bf = jnp.bfloat16
    h = jnp.dot(ef_ref[...].astype(bf), wa_ref[...],
                preferred_element_type=jnp.float32)
    h += jnp.dot(g_ref[...].astype(bf), wbc_ref[...],
                 preferred_element_type=jnp.float32)
    h += ce_ref[...]
    h = jnp.maximum(h, 0.0).astype(bf)
    e = jnp.dot(h, w2_ref[...], preferred_element_type=jnp.float32)
    e += b2_ref[...]
    e_ref[...] = e

    @pl.when(i == 0)
    def _():
        ecomb_ref[...] = jnp.zeros_like(ecomb_ref)

    ecomb_ref[...] += jnp.sum(e, axis=0, keepdims=True)


def _node_body(nf_ref, hp0_ref, hp1_ref, hp2_ref, hp3_ref, hp4_ref,
               ecomb_ref, wnn_ref, wnh_ref, cn_ref,
               wn2_ref, bn2_ref, wgn_ref, wge_ref, cu_ref, wu2_ref, bu2_ref,
               n_ref, g_ref, nacc_ref):
    i = pl.program_id(0)
    bf = jnp.bfloat16
    hsum = (hp0_ref[0] + hp0_ref[1] + hp1_ref[0] + hp1_ref[1]
            + hp2_ref[0] + hp2_ref[1] + hp3_ref[0] + hp3_ref[1]
            + hp4_ref[0] + hp4_ref[1])
    x = jnp.dot(nf_ref[...].astype(bf), wnn_ref[...],
                preferred_element_type=jnp.float32)
    x += jnp.dot(hsum.astype(bf), wnh_ref[...],
                 preferred_element_type=jnp.float32)
    x += cn_ref[...]
    x = jnp.maximum(x, 0.0).astype(bf)
    n = jnp.dot(x, wn2_ref[...], preferred_element_type=jnp.float32)
    n += bn2_ref[...]
    n_ref[...] = n

    @pl.when(i == 0)
    def _():
        nacc_ref[...] = jnp.zeros_like(nacc_ref)

    nacc_ref[...] += jnp.sum(n, axis=0, keepdims=True)

    @pl.when(i == pl.num_programs(0) - 1)
    def _():
        g1 = jnp.dot(nacc_ref[...], wgn_ref[...],
                     preferred_element_type=jnp.float32)
        g1 += jnp.dot(ecomb_ref[...], wge_ref[...],
                      preferred_element_type=jnp.float32)
        g1 += cu_ref[...]
        g1 = jnp.maximum(g1, 0.0)
        g_ref[...] = jnp.dot(g1, wu2_ref[...],
                             preferred_element_type=jnp.float32) + bu2_ref[...]


def _const_spec(shape):
    return pl.BlockSpec(shape, lambda i: tuple(0 for _ in shape))


def _edge_mlp(edge_feat, g, wa, wbc, ce, w2, b2):
    ne = edge_feat.shape[0]
    grid = ne // BE
    return pl.pallas_call(
        _edge_body,
        grid=(grid,),
        in_specs=[
            pl.BlockSpec((BE, D_EDGE), lambda i: (i, 0)),
            pl.BlockSpec((BE, 2 * D_NODE), lambda i: (i, 0)),
            _const_spec((D_EDGE, HID)),
            _const_spec((2 * D_NODE, HID)),
            _const_spec((1, HID)),
            _const_spec((HID, D_OUT)),
            _const_spec((1, D_OUT)),
        ],
        out_specs=[
            pl.BlockSpec((BE, D_OUT), lambda i: (i, 0)),
            pl.BlockSpec((1, D_OUT), lambda i: (0, 0)),
        ],
        out_shape=[
            jax.ShapeDtypeStruct((ne, D_OUT), jnp.float32),
            jax.ShapeDtypeStruct((1, D_OUT), jnp.float32),
        ],
    )(edge_feat, g, wa, wbc, ce, w2, b2)


def _node_mlp(node_feat, hps, ecomb, wnn, wnh, cn, wn2, bn2,
              wgn, wge, cu, wu2, bu2):
    nn = node_feat.shape[0]
    grid = nn // BN
    hp_spec = pl.BlockSpec((_NC, BN, D_NODE), lambda i: (0, i, 0))
    return pl.pallas_call(
        _node_body,
        grid=(grid,),
        in_specs=[
            pl.BlockSpec((BN, D_NODE), lambda i: (i, 0)),
            hp_spec, hp_spec, hp_spec, hp_spec, hp_spec,
            _const_spec((1, D_OUT)),
            _const_spec((D_NODE, HID)),
            _const_spec((D_NODE, HID)),
            _const_spec((1, HID)),
            _const_spec((HID, D_OUT)),
            _const_spec((1, D_OUT)),
            _const_spec((D_OUT, HID)),
            _const_spec((D_OUT, HID)),
            _const_spec((1, HID)),
            _const_spec((HID, D_OUT)),
            _const_spec((1, D_OUT)),
        ],
        out_specs=[
            pl.BlockSpec((BN, D_OUT), lambda i: (i, 0)),
            pl.BlockSpec((1, D_OUT), lambda i: (0, 0)),
        ],
        out_shape=[
            jax.ShapeDtypeStruct((nn, D_OUT), jnp.float32),
            jax.ShapeDtypeStruct((1, D_OUT), jnp.float32),
        ],
        scratch_shapes=[pltpu.VMEM((1, D_OUT), jnp.float32)],
    )(node_feat, *hps, ecomb, wnn, wnh, cn, wn2, bn2,
      wgn, wge, cu, wu2, bu2)


def kernel(edge_index, edge_feat, node_feat, g_repr,
           W_edge, b_edge, W_edge2, b_edge2,
           W_node, b_node, W_node2, b_node2,
           W_u, b_u, W_u2, b_u2):
    bf = jnp.bfloat16
    src = edge_index[0]
    dst = edge_index[1]
    u = g_repr

    wa = W_edge[:, :D_EDGE].T.astype(bf)
    wbc = W_edge[:, D_EDGE:D_EDGE + 2 * D_NODE].T.astype(bf)
    ce = (b_edge + u[0] @ W_edge[:, D_EDGE + 2 * D_NODE:].T)[None]
    w2 = W_edge2.T.astype(bf)
    b2 = b_edge2[None]

    wnn = W_node[:, :D_NODE].T.astype(bf)
    wnh = W_node[:, D_NODE:2 * D_NODE].T.astype(bf)
    cn = (b_node + u[0] @ W_node[:, 2 * D_NODE:].T)[None]
    wn2 = W_node2.T.astype(bf)
    bn2 = b_node2[None]

    wgn = W_u[:, :D_OUT].T
    wge = W_u[:, D_OUT:2 * D_OUT].T
    cu = (b_u + u[0] @ W_u[:, 2 * D_OUT:].T)[None]
    wu2 = W_u2.T
    bu2 = b_u2[None]

    zeros_n = jnp.zeros((_NPAD, D_NODE), jnp.float32)

    e_chunks, ecombs, hps = [], [], []
    hp_prev = jnp.zeros((_NC, _NPAD, D_NODE), jnp.float32)
    for c in range(_C):
        sl = slice(c * _EC, (c + 1) * _EC)
        g = _sc_gather(node_feat, src[sl], dst[sl])
        e_c, ecomb_c = _edge_mlp(edge_feat[sl], g, wa, wbc, ce, w2, b2)
        dst_t = dst[sl].reshape(1, _NW, _NCH, _GCH)
        hp_prev = _sc_scatter1(e_c, dst_t, zeros_n, hp_prev)
        hps.append(hp_prev)
        e_chunks.append(e_c)
        ecombs.append(ecomb_c)

    e_new = jnp.concatenate(e_chunks, axis=0)
    e_comb = functools.reduce(lambda a, b: a + b, ecombs)

    n_new, g_new = _node_mlp(node_feat, hps, e_comb, wnn, wnh, cn,
                             wn2, bn2, wgn, wge, cu, wu2, bu2)
    return (e_new, n_new, g_new)


# speedup vs baseline: 1.0648x; 1.0263x over previous
import functools

import jax
import jax.numpy as jnp
from jax import lax
from jax.experimental import pallas as pl
from jax.experimental.pallas import tpu as pltpu
from jax.experimental.pallas import tpu_sc as plsc

N_NODES = 10000
N_EDGES = 320000
D_NODE = 128
D_EDGE = 16
D_U = 128
HID = 512
D_OUT = 128

_C = 5
_EC = N_EDGES // _C

BE = 2560
BN = 2000

_NC = 2
_NS = 16
_NW = _NC * _NS
_EPW = _EC // _NW
_GCH = 80
_NCH = _EPW // _GCH
_GK = 5
_GROWS = _GK * _GCH
_GOUT = _EPW // _GROWS

_SC_MESH = plsc.VectorSubcoreMesh(core_axis_name="c", subcore_axis_name="s")


_NPAD = 10240
_NPT = _NPAD // _NS


@functools.partial(
    pl.kernel,
    out_type=jax.ShapeDtypeStruct((_EC, 2 * D_NODE), jnp.float32),
    mesh=_SC_MESH,
    scratch_types=[
        pltpu.VMEM((_EPW,), jnp.int32),
        pltpu.VMEM((_EPW,), jnp.int32),
        pltpu.VMEM((_GROWS, D_NODE), jnp.float32),
        pltpu.VMEM((_GROWS, D_NODE), jnp.float32),
        pltpu.SemaphoreType.DMA,
        pltpu.SemaphoreType.DMA,
    ],
)
def _sc_gather(node_hbm, src_hbm, dst_hbm, g_hbm,
               src_v, dst_v, rs_v, rd_v, sem_g, sem_w):
    wid = lax.axis_index("s") * _NC + lax.axis_index("c")
    base = wid * _EPW
    pltpu.sync_copy(src_hbm.at[pl.ds(base, _EPW)], src_v)
    pltpu.sync_copy(dst_hbm.at[pl.ds(base, _EPW)], dst_v)

    def body(j, carry):
        off = j * _GROWS
        copies = []
        for b in range(_GK):
            o = b * _GCH
            copies.append(pltpu.async_copy(
                node_hbm.at[src_v.at[pl.ds(off + o, _GCH)]],
                rs_v.at[pl.ds(o, _GCH)], sem_g))
            copies.append(pltpu.async_copy(
                node_hbm.at[dst_v.at[pl.ds(off + o, _GCH)]],
                rd_v.at[pl.ds(o, _GCH)], sem_g))
        for c in copies:
            c.wait()
        ws = pltpu.async_copy(
            rs_v, g_hbm.at[pl.ds(base + off, _GROWS), pl.ds(0, D_NODE)],
            sem_w)
        wd = pltpu.async_copy(
            rd_v, g_hbm.at[pl.ds(base + off, _GROWS), pl.ds(D_NODE, D_NODE)],
            sem_w)
        ws.wait()
        wd.wait()
        return carry

    lax.fori_loop(0, _GOUT, body, 0)


def _scatter_work(e_refs, dst4_ref, zeros_ref, hp_ref, idx_v, rows_v, acc_sh):
    k = len(e_refs)
    cid = lax.axis_index("c")
    sid = lax.axis_index("s")
    wid = sid * _NC + cid
    base = wid * _EPW
    row0 = sid * _NPT
    pltpu.sync_copy(zeros_ref.at[pl.ds(row0, _NPT)],
                    acc_sh.at[pl.ds(row0, _NPT)])
    for ci in range(k):
        pltpu.sync_copy(dst4_ref.at[ci, wid],
                        idx_v.at[pl.ds(ci * _NCH, _NCH)])
    plsc.subcore_barrier()

    for ci in range(k):
        def body(j, carry):
            pltpu.sync_copy(e_refs[ci].at[pl.ds(base + j * _GCH, _GCH)],
                            rows_v)
            pltpu.sync_copy(rows_v, acc_sh.at[idx_v.at[ci * _NCH + j]],
                            add=True)
            return carry

        lax.fori_loop(0, _NCH, body, 0)

    plsc.subcore_barrier()
    pltpu.sync_copy(acc_sh.at[pl.ds(row0, _NPT)],
                    hp_ref.at[cid, pl.ds(row0, _NPT)])


def _scatter_types(k):
    return dict(
        out_type=jax.ShapeDtypeStruct((_NC, _NPAD, D_NODE), jnp.float32),
        mesh=_SC_MESH,
        scratch_types=[
            pltpu.VMEM((k * _NCH, _GCH), jnp.int32),
            pltpu.VMEM((_GCH, D_NODE), jnp.float32),
            pltpu.VMEM_SHARED((_NPAD, D_NODE), jnp.float32),
        ],
    )


@functools.partial(pl.kernel, **_scatter_types(1))
def _sc_scatter1(e0, dst4, zeros_hbm, prev_hbm, hp, idx_v, rows_v, acc_sh):
    _scatter_work([e0], dst4, zeros_hbm, hp, idx_v, rows_v, acc_sh)


@functools.partial(pl.kernel, **_scatter_types(2))
def _sc_scatter2(e0, e1, dst4, zeros_hbm, prev_hbm, hp, idx_v, rows_v,
                 acc_sh):
    _scatter_work([e0, e1], dst4, zeros_hbm, hp, idx_v, rows_v, acc_sh)


def _edge_body(ef_ref, g_ref, wa_ref, wbc_ref, ce_ref,
               w2_ref, b2_ref, e_ref, ecomb_ref):
    i = pl.program_id(0)
    bf = jnp.bfloat16
    h = jnp.dot(ef_ref[...].astype(bf), wa_ref[...],
                preferred_element_type=jnp.float32)
    h += jnp.dot(g_ref[...].astype(bf), wbc_ref[...],
                 preferred_element_type=jnp.float32)
    h += ce_ref[...]
    h = jnp.maximum(h, 0.0).astype(bf)
    e = jnp.dot(h, w2_ref[...], preferred_element_type=jnp.float32)
    e += b2_ref[...]
    e_ref[...] = e

    @pl.when(i == 0)
    def _():
        ecomb_ref[...] = jnp.zeros_like(ecomb_ref)

    ecomb_ref[...] += jnp.sum(e, axis=0, keepdims=True)


def _node_body(nf_ref, hp0_ref, hp1_ref, hp2_ref,
               ecomb_ref, wnn_ref, wnh_ref, cn_ref,
               wn2_ref, bn2_ref, wgn_ref, wge_ref, cu_ref, wu2_ref, bu2_ref,
               n_ref, g_ref, nacc_ref):
    i = pl.program_id(0)
    bf = jnp.bfloat16
    hsum = (hp0_ref[0] + hp0_ref[1] + hp1_ref[0] + hp1_ref[1]
            + hp2_ref[0] + hp2_ref[1])
    x = jnp.dot(nf_ref[...].astype(bf), wnn_ref[...],
                preferred_element_type=jnp.float32)
    x += jnp.dot(hsum.astype(bf), wnh_ref[...],
                 preferred_element_type=jnp.float32)
    x += cn_ref[...]
    x = jnp.maximum(x, 0.0).astype(bf)
    n = jnp.dot(x, wn2_ref[...], preferred_element_type=jnp.float32)
    n += bn2_ref[...]
    n_ref[...] = n

    @pl.when(i == 0)
    def _():
        nacc_ref[...] = jnp.zeros_like(nacc_ref)

    nacc_ref[...] += jnp.sum(n, axis=0, keepdims=True)

    @pl.when(i == pl.num_programs(0) - 1)
    def _():
        g1 = jnp.dot(nacc_ref[...], wgn_ref[...],
                     preferred_element_type=jnp.float32)
        g1 += jnp.dot(ecomb_ref[...], wge_ref[...],
                      preferred_element_type=jnp.float32)
        g1 += cu_ref[...]
        g1 = jnp.maximum(g1, 0.0)
        g_ref[...] = jnp.dot(g1, wu2_ref[...],
                             preferred_element_type=jnp.float32) + bu2_ref[...]


def _const_spec(shape):
    return pl.BlockSpec(shape, lambda i: tuple(0 for _ in shape))


def _edge_mlp(edge_feat, g, wa, wbc, ce, w2, b2):
    ne = edge_feat.shape[0]
    grid = ne // BE
    return pl.pallas_call(
        _edge_body,
        grid=(grid,),
        in_specs=[
            pl.BlockSpec((BE, D_EDGE), lambda i: (i, 0)),
            pl.BlockSpec((BE, 2 * D_NODE), lambda i: (i, 0)),
            _const_spec((D_EDGE, HID)),
            _const_spec((2 * D_NODE, HID)),
            _const_spec((1, HID)),
            _const_spec((HID, D_OUT)),
            _const_spec((1, D_OUT)),
        ],
        out_specs=[
            pl.BlockSpec((BE, D_OUT), lambda i: (i, 0)),
            pl.BlockSpec((1, D_OUT), lambda i: (0, 0)),
        ],
        out_shape=[
            jax.ShapeDtypeStruct((ne, D_OUT), jnp.float32),
            jax.ShapeDtypeStruct((1, D_OUT), jnp.float32),
        ],
    )(edge_feat, g, wa, wbc, ce, w2, b2)


def _node_mlp(node_feat, hps, ecomb, wnn, wnh, cn, wn2, bn2,
              wgn, wge, cu, wu2, bu2):
    nn = node_feat.shape[0]
    grid = nn // BN
    hp_spec = pl.BlockSpec((_NC, BN, D_NODE), lambda i: (0, i, 0))
    return pl.pallas_call(
        _node_body,
        grid=(grid,),
        in_specs=[
            pl.BlockSpec((BN, D_NODE), lambda i: (i, 0)),
            hp_spec, hp_spec, hp_spec,
            _const_spec((1, D_OUT)),
            _const_spec((D_NODE, HID)),
            _const_spec((D_NODE, HID)),
            _const_spec((1, HID)),
            _const_spec((HID, D_OUT)),
            _const_spec((1, D_OUT)),
            _const_spec((D_OUT, HID)),
            _const_spec((D_OUT, HID)),
            _const_spec((1, HID)),
            _const_spec((HID, D_OUT)),
            _const_spec((1, D_OUT)),
        ],
        out_specs=[
            pl.BlockSpec((BN, D_OUT), lambda i: (i, 0)),
            pl.BlockSpec((1, D_OUT), lambda i: (0, 0)),
        ],
        out_shape=[
            jax.ShapeDtypeStruct((nn, D_OUT), jnp.float32),
            jax.ShapeDtypeStruct((1, D_OUT), jnp.float32),
        ],
        scratch_shapes=[pltpu.VMEM((1, D_OUT), jnp.float32)],
    )(node_feat, *hps, ecomb, wnn, wnh, cn, wn2, bn2,
      wgn, wge, cu, wu2, bu2)


def kernel(edge_index, edge_feat, node_feat, g_repr,
           W_edge, b_edge, W_edge2, b_edge2,
           W_node, b_node, W_node2, b_node2,
           W_u, b_u, W_u2, b_u2):
    bf = jnp.bfloat16
    src = edge_index[0]
    dst = edge_index[1]
    u = g_repr

    wa = W_edge[:, :D_EDGE].T.astype(bf)
    wbc = W_edge[:, D_EDGE:D_EDGE + 2 * D_NODE].T.astype(bf)
    ce = (b_edge + u[0] @ W_edge[:, D_EDGE + 2 * D_NODE:].T)[None]
    w2 = W_edge2.T.astype(bf)
    b2 = b_edge2[None]

    wnn = W_node[:, :D_NODE].T.astype(bf)
    wnh = W_node[:, D_NODE:2 * D_NODE].T.astype(bf)
    cn = (b_node + u[0] @ W_node[:, 2 * D_NODE:].T)[None]
    wn2 = W_node2.T.astype(bf)
    bn2 = b_node2[None]

    wgn = W_u[:, :D_OUT].T
    wge = W_u[:, D_OUT:2 * D_OUT].T
    cu = (b_u + u[0] @ W_u[:, 2 * D_OUT:].T)[None]
    wu2 = W_u2.T
    bu2 = b_u2[None]

    zeros_n = jnp.zeros((_NPAD, D_NODE), jnp.float32)

    e_chunks, ecombs = [], []
    for c in range(_C):
        sl = slice(c * _EC, (c + 1) * _EC)
        g = _sc_gather(node_feat, src[sl], dst[sl])
        e_c, ecomb_c = _edge_mlp(edge_feat[sl], g, wa, wbc, ce, w2, b2)
        e_chunks.append(e_c)
        ecombs.append(ecomb_c)

    hps = []
    hp_prev = jnp.zeros((_NC, _NPAD, D_NODE), jnp.float32)
    dst_a = dst[0:2 * _EC].reshape(2, _NW, _NCH, _GCH)
    dst_b = dst[2 * _EC:4 * _EC].reshape(2, _NW, _NCH, _GCH)
    dst_c = dst[4 * _EC:].reshape(1, _NW, _NCH, _GCH)
    hp_prev = _sc_scatter2(e_chunks[0], e_chunks[1], dst_a, zeros_n, hp_prev)
    hps.append(hp_prev)
    hp_prev = _sc_scatter2(e_chunks[2], e_chunks[3], dst_b, zeros_n, hp_prev)
    hps.append(hp_prev)
    hp_prev = _sc_scatter1(e_chunks[4], dst_c, zeros_n, hp_prev)
    hps.append(hp_prev)

    e_new = jnp.concatenate(e_chunks, axis=0)
    e_comb = functools.reduce(lambda a, b: a + b, ecombs)

    n_new, g_new = _node_mlp(node_feat, hps, e_comb, wnn, wnh, cn,
                             wn2, bn2, wgn, wge, cu, wu2, bu2)
    return (e_new, n_new, g_new)
